# baseline (speedup 1.0000x reference)
import jax
import jax.numpy as jnp
from jax import lax
from jax.experimental import pallas as pl
from jax.experimental.pallas import tpu as pltpu
from jax.experimental.pallas import tpu_sc as plsc

BATCH = 4096
SEQ = 200
EMBED = 64
LANES = 16

NUM_CORES = 2
NUM_SUBCORES = 16
NW = NUM_CORES * NUM_SUBCORES
B_PER_W = BATCH // NW
HALF = SEQ // 2
IDX_ROWS = 2 * B_PER_W
NVREG = EMBED // LANES
INV_SEQ = 1.0 / SEQ


def _sc_body(x_hbm, table_hbm, out_hbm, idx_v, buf0, buf1, out_v, sem0, sem1):
    wid = lax.axis_index("s") * NUM_CORES + lax.axis_index("c")
    ibase = wid * IDX_ROWS
    obase = wid * B_PER_W

    pltpu.sync_copy(x_hbm.at[pl.ds(ibase, IDX_ROWS)], idx_v)

    def start(b, buf, sem):
        pltpu.async_copy(table_hbm.at[idx_v.at[2 * b]], buf.at[pl.ds(0, HALF)], sem)
        pltpu.async_copy(table_hbm.at[idx_v.at[2 * b + 1]], buf.at[pl.ds(HALF, HALF)], sem)

    def wait(buf, sem):
        pltpu.make_async_copy(table_hbm.at[idx_v.at[0]], buf.at[pl.ds(0, HALF)], sem).wait()
        pltpu.make_async_copy(table_hbm.at[idx_v.at[0]], buf.at[pl.ds(HALF, HALF)], sem).wait()

    def accum_store(b, buf):
        def rbody(r4, acc):
            r = r4 * 4
            out = []
            for k in range(NVREG):
                s = buf[r, pl.ds(LANES * k, LANES)] + buf[r + 1, pl.ds(LANES * k, LANES)]
                t = buf[r + 2, pl.ds(LANES * k, LANES)] + buf[r + 3, pl.ds(LANES * k, LANES)]
                out.append(acc[k] + (s + t))
            return tuple(out)

        zero = jnp.zeros((LANES,), jnp.float32)
        acc = lax.fori_loop(0, SEQ // 4, rbody, (zero,) * NVREG)
        for k in range(NVREG):
            out_v[b, pl.ds(LANES * k, LANES)] = acc[k] * INV_SEQ

    start(0, buf0, sem0)

    def body(t, _):
        b0 = 2 * t
        start(b0 + 1, buf1, sem1)
        wait(buf0, sem0)
        accum_store(b0, buf0)

        @pl.when(b0 + 2 < B_PER_W)
        def _():
            start(b0 + 2, buf0, sem0)

        wait(buf1, sem1)
        accum_store(b0 + 1, buf1)
        return 0

    lax.fori_loop(0, B_PER_W // 2, body, 0)

    pltpu.sync_copy(out_v, out_hbm.at[pl.ds(obase, B_PER_W)])


@jax.jit
def _encode(x2, table):
    mesh = plsc.VectorSubcoreMesh(core_axis_name="c", subcore_axis_name="s")
    return pl.kernel(
        _sc_body,
        out_type=jax.ShapeDtypeStruct((BATCH, EMBED), jnp.float32),
        mesh=mesh,
        compiler_params=pltpu.CompilerParams(use_tc_tiling_on_sc=False),
        scratch_types=[
            pltpu.VMEM((IDX_ROWS, HALF), jnp.int32),
            pltpu.VMEM((SEQ, EMBED), jnp.float32),
            pltpu.VMEM((SEQ, EMBED), jnp.float32),
            pltpu.VMEM((B_PER_W, EMBED), jnp.float32),
            pltpu.SemaphoreType.DMA,
            pltpu.SemaphoreType.DMA,
        ],
    )(x2, table)


def kernel(x, table):
    x2 = x.astype(jnp.int32).reshape(BATCH * 2, HALF)
    return _encode(x2, table)
